# baseline (device time: 51352 ns/iter reference)
import os

import jax
import jax.numpy as jnp
from jax import lax
from jax.experimental import pallas as pl
from jax.experimental.pallas import tpu as pltpu

_NOCOMM = bool(int(os.environ.get("NOCOMM", "0")))

N_DEV = 8
SQ = 512
D = 1024
HQ = 8
DH = 128
R = SQ // N_DEV
KV_CHUNK = 1024
SCALE = 0.08838834764831843
CW = D + 128
M_OFF = D


def kernel(x, Wq, Wo, K_ext, V_ext):
    skv = K_ext.shape[1]
    x2 = x.reshape(SQ, D).astype(jnp.bfloat16)
    wq_bf = Wq.astype(jnp.bfloat16)
    k2 = K_ext.reshape(skv, HQ * DH).astype(jnp.bfloat16)
    v2 = V_ext.reshape(skv, HQ * DH).astype(jnp.bfloat16)

    def body(x_ref, wq_ref, wo_ref, k_ref, v_ref, out_ref,
             q_ref, acc_ref, rs_ostage, rs_obuf, rs_mlbuf, ag_stage, ag_buf,
             rs_osend_sems, rs_orecv_sems, rs_mlsend_sems, rs_mlrecv_sems,
             ag_send_sems, ag_recv_sems):
        my = lax.axis_index("i")

        if not _NOCOMM:
            barrier_sem = pltpu.get_barrier_semaphore()
            for j in range(1, N_DEV):
                pl.semaphore_signal(
                    barrier_sem, inc=1,
                    device_id=(lax.rem(my + j, N_DEV),),
                    device_id_type=pl.DeviceIdType.MESH,
                )

        q_ref[...] = (
            jnp.dot(x_ref[...], wq_ref[...],
                    preferred_element_type=jnp.float32) * SCALE
        ).astype(jnp.bfloat16)

        n_chunks = skv // KV_CHUNK
        rs_o_rdmas = []
        rows = pl.ds(my * R, R)

        def merge_head(g):
            cg = slice(g * DH, (g + 1) * DH)
            for j in range(1, N_DEV):
                s = j - 1
                rs_o_rdmas[g * (N_DEV - 1) + s].wait_recv()
                acc_ref[rows, cg] = (
                    acc_ref[rows, cg]
                    + rs_obuf[s, :, cg].astype(jnp.float32)
                )

        for h in range(HQ):
            c = slice(h * DH, (h + 1) * DH)
            qh = q_ref[:, c]

            def chunk_body(ci, carry, c=c, qh=qh):
                l, u = carry
                s = lax.dot_general(
                    qh, k_ref[pl.ds(ci * KV_CHUNK, KV_CHUNK), c],
                    (((1,), (1,)), ((), ())),
                    preferred_element_type=jnp.float32,
                )
                p = jnp.exp(s.astype(jnp.bfloat16))
                ones_col = jnp.ones((KV_CHUNK, 1), jnp.bfloat16)
                l = l + jnp.dot(
                    p, ones_col, preferred_element_type=jnp.float32)
                u = u + jnp.dot(
                    p, v_ref[pl.ds(ci * KV_CHUNK, KV_CHUNK), c],
                    preferred_element_type=jnp.float32)
                return l, u

            l0 = jnp.zeros((SQ, 1), dtype=jnp.float32)
            u0 = jnp.zeros((SQ, DH), dtype=jnp.float32)
            l, u = lax.fori_loop(0, n_chunks, chunk_body, (l0, u0))
            acc_ref[:, c] = u
            acc_ref[:, M_OFF + h:M_OFF + h + 1] = l

            if h == 0 and not _NOCOMM:
                pl.semaphore_wait(barrier_sem, N_DEV - 1)
            for j in range(1, N_DEV) if not _NOCOMM else ():
                tgt = lax.rem(my + j, N_DEV)
                trows = pl.ds(tgt * R, R)
                rs_ostage[j - 1, :, c] = (
                    acc_ref[trows, c].astype(jnp.bfloat16)
                )
                o_rdma = pltpu.make_async_remote_copy(
                    src_ref=rs_ostage.at[j - 1, :, pl.ds(h * DH, DH)],
                    dst_ref=rs_obuf.at[j - 1, :, pl.ds(h * DH, DH)],
                    send_sem=rs_osend_sems.at[j - 1, h],
                    recv_sem=rs_orecv_sems.at[j - 1, h],
                    device_id=(tgt,),
                    device_id_type=pl.DeviceIdType.MESH,
                )
                o_rdma.start()
                rs_o_rdmas.append(o_rdma)
            if h >= 2 and not _NOCOMM:
                merge_head(h - 2)

        rs_ml_rdmas = []
        for j in range(1, N_DEV) if not _NOCOMM else ():
            tgt = lax.rem(my + j, N_DEV)
            ml_rdma = pltpu.make_async_remote_copy(
                src_ref=acc_ref.at[pl.ds(tgt * R, R), pl.ds(M_OFF, DH)],
                dst_ref=rs_mlbuf.at[j - 1],
                send_sem=rs_mlsend_sems.at[j - 1],
                recv_sem=rs_mlrecv_sems.at[j - 1],
                device_id=(tgt,),
                device_id_type=pl.DeviceIdType.MESH,
            )
            ml_rdma.start()
            rs_ml_rdmas.append(ml_rdma)

        if not _NOCOMM:
            merge_head(HQ - 2)
            merge_head(HQ - 1)
        for j in range(1, N_DEV) if not _NOCOMM else ():
            s = j - 1
            rs_ml_rdmas[s].wait_recv()
            acc_ref[rows, M_OFF:M_OFF + HQ] = (
                acc_ref[rows, M_OFF:M_OFF + HQ] + rs_mlbuf[s, :, :HQ]
            )

        for h in range(HQ):
            c = slice(h * DH, (h + 1) * DH)
            acc_ref[rows, c] = (
                acc_ref[rows, c] / acc_ref[rows, M_OFF + h:M_OFF + h + 1]
            )
        out_ref[0, rows, :] = jnp.dot(
            acc_ref[rows, :D], wo_ref[...], preferred_element_type=jnp.float32
        )

        ag_rdmas = []
        if not _NOCOMM:
            ag_stage[...] = out_ref[0, rows, :].astype(jnp.bfloat16)
        for j in range(1, N_DEV) if not _NOCOMM else ():
            tgt = lax.rem(my + j, N_DEV)
            rdma = pltpu.make_async_remote_copy(
                src_ref=ag_stage,
                dst_ref=ag_buf.at[j - 1],
                send_sem=ag_send_sems.at[j - 1],
                recv_sem=ag_recv_sems.at[j - 1],
                device_id=(tgt,),
                device_id_type=pl.DeviceIdType.MESH,
            )
            rdma.start()
            ag_rdmas.append(rdma)

        for rdma in rs_o_rdmas:
            rdma.wait_send()
        for rdma in rs_ml_rdmas:
            rdma.wait_send()
        for j in range(1, N_DEV) if not _NOCOMM else ():
            s = j - 1
            ag_rdmas[s].wait_recv()
            src_dev = lax.rem(my - j + N_DEV, N_DEV)
            out_ref[0, pl.ds(src_dev * R, R), :] = (
                ag_buf[s].astype(jnp.float32)
            )
        for rdma in ag_rdmas:
            rdma.wait_send()

    out = pl.pallas_call(
        body,
        out_shape=jax.ShapeDtypeStruct((1, SQ, D), jnp.float32),
        in_specs=[pl.BlockSpec(memory_space=pltpu.VMEM)] * 5,
        out_specs=pl.BlockSpec(memory_space=pltpu.VMEM),
        scratch_shapes=[
            pltpu.VMEM((SQ, D), jnp.bfloat16),
            pltpu.VMEM((SQ, CW), jnp.float32),
            pltpu.VMEM((N_DEV - 1, R, D), jnp.bfloat16),
            pltpu.VMEM((N_DEV - 1, R, D), jnp.bfloat16),
            pltpu.VMEM((N_DEV - 1, R, DH), jnp.float32),
            pltpu.VMEM((R, D), jnp.bfloat16),
            pltpu.VMEM((N_DEV - 1, R, D), jnp.bfloat16),
            pltpu.SemaphoreType.DMA((N_DEV - 1, HQ)),
            pltpu.SemaphoreType.DMA((N_DEV - 1, HQ)),
            pltpu.SemaphoreType.DMA((N_DEV - 1,)),
            pltpu.SemaphoreType.DMA((N_DEV - 1,)),
            pltpu.SemaphoreType.DMA((N_DEV - 1,)),
            pltpu.SemaphoreType.DMA((N_DEV - 1,)),
        ],
        compiler_params=(
            pltpu.CompilerParams()
            if _NOCOMM
            else pltpu.CompilerParams(collective_id=0)
        ),
    )(x2, wq_bf, Wo, k2, v2)
    return out


# device time: 51259 ns/iter; 1.0018x vs baseline; 1.0018x over previous
import os

import jax
import jax.numpy as jnp
from jax import lax
from jax.experimental import pallas as pl
from jax.experimental.pallas import tpu as pltpu

_NOCOMM = bool(int(os.environ.get("NOCOMM", "0")))

N_DEV = 8
SQ = 512
D = 1024
HQ = 8
DH = 128
R = SQ // N_DEV
KV_CHUNK = 1024
SCALE = 0.08838834764831843
CW = D + 128
M_OFF = D


def kernel(x, Wq, Wo, K_ext, V_ext):
    skv = K_ext.shape[1]
    x2 = x.reshape(SQ, D).astype(jnp.bfloat16)
    wq_bf = Wq.astype(jnp.bfloat16)
    k2 = K_ext.reshape(skv, HQ * DH).astype(jnp.bfloat16)
    v2 = V_ext.reshape(skv, HQ * DH).astype(jnp.bfloat16)

    def body(x_ref, wq_ref, wo_ref, k_ref, v_ref, out_ref,
             q_ref, acc_ref, rs_ostage, rs_obuf, rs_mlbuf, ag_stage, ag_buf,
             rs_osend_sems, rs_orecv_sems, rs_mlsend_sems, rs_mlrecv_sems,
             ag_send_sems, ag_recv_sems):
        my = lax.axis_index("i")

        if not _NOCOMM:
            barrier_sem = pltpu.get_barrier_semaphore()
            for j in range(1, N_DEV):
                pl.semaphore_signal(
                    barrier_sem, inc=1,
                    device_id=(lax.rem(my + j, N_DEV),),
                    device_id_type=pl.DeviceIdType.MESH,
                )

        q_ref[...] = (
            jnp.dot(x_ref[...], wq_ref[...],
                    preferred_element_type=jnp.float32) * SCALE
        ).astype(jnp.bfloat16)

        n_chunks = skv // KV_CHUNK
        rs_o_rdmas = []
        rows = pl.ds(my * R, R)

        def merge_head(g):
            cg = slice(g * DH, (g + 1) * DH)
            for j in range(1, N_DEV):
                s = j - 1
                rs_o_rdmas[g * (N_DEV - 1) + s].wait_recv()
                acc_ref[rows, cg] = (
                    acc_ref[rows, cg]
                    + rs_obuf[s, :, cg].astype(jnp.float32)
                )

        for h in range(HQ):
            c = slice(h * DH, (h + 1) * DH)
            qh = q_ref[:, c]

            def chunk_body(ci, carry, c=c, qh=qh):
                l, u = carry
                s = lax.dot_general(
                    qh, k_ref[pl.ds(ci * KV_CHUNK, KV_CHUNK), c],
                    (((1,), (1,)), ((), ())),
                    preferred_element_type=jnp.float32,
                )
                p = jnp.exp(s).astype(jnp.bfloat16)
                ones_col = jnp.ones((KV_CHUNK, 1), jnp.bfloat16)
                l = l + jnp.dot(
                    p, ones_col, preferred_element_type=jnp.float32)
                u = u + jnp.dot(
                    p, v_ref[pl.ds(ci * KV_CHUNK, KV_CHUNK), c],
                    preferred_element_type=jnp.float32)
                return l, u

            l0 = jnp.zeros((SQ, 1), dtype=jnp.float32)
            u0 = jnp.zeros((SQ, DH), dtype=jnp.float32)
            l, u = lax.fori_loop(0, n_chunks, chunk_body, (l0, u0))
            acc_ref[:, c] = u
            acc_ref[:, M_OFF + h:M_OFF + h + 1] = l

            if h == 0 and not _NOCOMM:
                pl.semaphore_wait(barrier_sem, N_DEV - 1)
            for j in range(1, N_DEV) if not _NOCOMM else ():
                tgt = lax.rem(my + j, N_DEV)
                trows = pl.ds(tgt * R, R)
                rs_ostage[j - 1, :, c] = (
                    acc_ref[trows, c].astype(jnp.bfloat16)
                )
                o_rdma = pltpu.make_async_remote_copy(
                    src_ref=rs_ostage.at[j - 1, :, pl.ds(h * DH, DH)],
                    dst_ref=rs_obuf.at[j - 1, :, pl.ds(h * DH, DH)],
                    send_sem=rs_osend_sems.at[j - 1, h],
                    recv_sem=rs_orecv_sems.at[j - 1, h],
                    device_id=(tgt,),
                    device_id_type=pl.DeviceIdType.MESH,
                )
                o_rdma.start()
                rs_o_rdmas.append(o_rdma)
            if h >= 2 and not _NOCOMM:
                merge_head(h - 2)

        rs_ml_rdmas = []
        for j in range(1, N_DEV) if not _NOCOMM else ():
            tgt = lax.rem(my + j, N_DEV)
            ml_rdma = pltpu.make_async_remote_copy(
                src_ref=acc_ref.at[pl.ds(tgt * R, R), pl.ds(M_OFF, DH)],
                dst_ref=rs_mlbuf.at[j - 1],
                send_sem=rs_mlsend_sems.at[j - 1],
                recv_sem=rs_mlrecv_sems.at[j - 1],
                device_id=(tgt,),
                device_id_type=pl.DeviceIdType.MESH,
            )
            ml_rdma.start()
            rs_ml_rdmas.append(ml_rdma)

        if not _NOCOMM:
            merge_head(HQ - 2)
            merge_head(HQ - 1)
        for j in range(1, N_DEV) if not _NOCOMM else ():
            s = j - 1
            rs_ml_rdmas[s].wait_recv()
            acc_ref[rows, M_OFF:M_OFF + HQ] = (
                acc_ref[rows, M_OFF:M_OFF + HQ] + rs_mlbuf[s, :, :HQ]
            )

        for h in range(HQ):
            c = slice(h * DH, (h + 1) * DH)
            acc_ref[rows, c] = (
                acc_ref[rows, c] / acc_ref[rows, M_OFF + h:M_OFF + h + 1]
            )
        out_ref[0, rows, :] = jnp.dot(
            acc_ref[rows, :D], wo_ref[...], preferred_element_type=jnp.float32
        )

        ag_rdmas = []
        if not _NOCOMM:
            ag_stage[...] = out_ref[0, rows, :].astype(jnp.bfloat16)
        for j in range(1, N_DEV) if not _NOCOMM else ():
            tgt = lax.rem(my + j, N_DEV)
            rdma = pltpu.make_async_remote_copy(
                src_ref=ag_stage,
                dst_ref=ag_buf.at[j - 1],
                send_sem=ag_send_sems.at[j - 1],
                recv_sem=ag_recv_sems.at[j - 1],
                device_id=(tgt,),
                device_id_type=pl.DeviceIdType.MESH,
            )
            rdma.start()
            ag_rdmas.append(rdma)

        for rdma in rs_o_rdmas:
            rdma.wait_send()
        for rdma in rs_ml_rdmas:
            rdma.wait_send()
        for j in range(1, N_DEV) if not _NOCOMM else ():
            s = j - 1
            ag_rdmas[s].wait_recv()
            src_dev = lax.rem(my - j + N_DEV, N_DEV)
            out_ref[0, pl.ds(src_dev * R, R), :] = (
                ag_buf[s].astype(jnp.float32)
            )
        for rdma in ag_rdmas:
            rdma.wait_send()

    out = pl.pallas_call(
        body,
        out_shape=jax.ShapeDtypeStruct((1, SQ, D), jnp.float32),
        in_specs=[pl.BlockSpec(memory_space=pltpu.VMEM)] * 5,
        out_specs=pl.BlockSpec(memory_space=pltpu.VMEM),
        scratch_shapes=[
            pltpu.VMEM((SQ, D), jnp.bfloat16),
            pltpu.VMEM((SQ, CW), jnp.float32),
            pltpu.VMEM((N_DEV - 1, R, D), jnp.bfloat16),
            pltpu.VMEM((N_DEV - 1, R, D), jnp.bfloat16),
            pltpu.VMEM((N_DEV - 1, R, DH), jnp.float32),
            pltpu.VMEM((R, D), jnp.bfloat16),
            pltpu.VMEM((N_DEV - 1, R, D), jnp.bfloat16),
            pltpu.SemaphoreType.DMA((N_DEV - 1, HQ)),
            pltpu.SemaphoreType.DMA((N_DEV - 1, HQ)),
            pltpu.SemaphoreType.DMA((N_DEV - 1,)),
            pltpu.SemaphoreType.DMA((N_DEV - 1,)),
            pltpu.SemaphoreType.DMA((N_DEV - 1,)),
            pltpu.SemaphoreType.DMA((N_DEV - 1,)),
        ],
        compiler_params=(
            pltpu.CompilerParams()
            if _NOCOMM
            else pltpu.CompilerParams(collective_id=0)
        ),
    )(x2, wq_bf, Wo, k2, v2)
    return out
